# Initial kernel scaffold; baseline (speedup 1.0000x reference)
import jax
import jax.numpy as jnp
from jax.experimental import pallas as pl


def kernel(pointsI, pointsJ, wi, wj):
    raise NotImplementedError("write your pallas kernel here")


# speedup vs baseline: 1.0339x; 1.0339x over previous
import jax
import jax.numpy as jnp
from jax.experimental import pallas as pl
from jax.experimental.pallas import tpu as pltpu

_BI = 256


def _nn_kernel(iblk_ref, jt_ref, idx1_ref, idx2_ref, s1_ref, s2_ref,
               colmin_ref, colabs_ref, acc_ref):
    i = pl.program_id(0)
    nsteps = pl.num_programs(0)
    bi = iblk_ref.shape[0]
    n = jt_ref.shape[1]

    iblk = iblk_ref[...]
    jt = jt_ref[...]
    xi = iblk[:, 0:1]
    yi = iblk[:, 1:2]
    xj = jt[0:1, :]
    yj = jt[1:2, :]

    a2 = jnp.sum(iblk * iblk, axis=1, keepdims=True)
    b2 = jnp.sum(jt * jt, axis=0, keepdims=True)
    mm = jax.lax.dot_general(iblk, jt, (((1,), (0,)), ((), ())),
                             preferred_element_type=jnp.float32)
    dd = (a2 + b2) - 2.0 * mm
    d2 = jnp.sqrt(jnp.maximum(dd, 0.0))

    jcol = jax.lax.broadcasted_iota(jnp.int32, (bi, n), 1)
    irow = jax.lax.broadcasted_iota(jnp.int32, (bi, n), 0)

    rmin = jnp.min(d2, axis=1, keepdims=True)
    idx1 = jnp.min(jnp.where(d2 == rmin, jcol, n), axis=1, keepdims=True)
    sel1 = jcol == idx1
    mjx = jnp.sum(jnp.where(sel1, xj, 0.0), axis=1, keepdims=True)
    mjy = jnp.sum(jnp.where(sel1, yj, 0.0), axis=1, keepdims=True)
    rowabs = jnp.abs(xi - mjx) + jnp.abs(yi - mjy)
    idx1_ref[...] = idx1

    cmin = jnp.min(d2, axis=0, keepdims=True)
    cloc = jnp.min(jnp.where(d2 == cmin, irow, bi), axis=0, keepdims=True)
    sel2 = irow == cloc
    mix = jnp.sum(jnp.where(sel2, xi, 0.0), axis=0, keepdims=True)
    miy = jnp.sum(jnp.where(sel2, yi, 0.0), axis=0, keepdims=True)
    cabs = jnp.abs(xj - mix) + jnp.abs(yj - miy)
    cidx = cloc + i * bi

    @pl.when(i == 0)
    def _init():
        colmin_ref[...] = jnp.full((1, n), jnp.inf, jnp.float32)
        idx2_ref[...] = jnp.zeros((1, n), jnp.int32)
        colabs_ref[...] = jnp.zeros((1, n), jnp.float32)
        acc_ref[0] = 0.0

    upd = cmin < colmin_ref[...]
    colmin_ref[...] = jnp.where(upd, cmin, colmin_ref[...])
    idx2_ref[...] = jnp.where(upd, cidx, idx2_ref[...])
    colabs_ref[...] = jnp.where(upd, cabs, colabs_ref[...])
    acc_ref[0] += jnp.sum(rowabs)

    @pl.when(i == nsteps - 1)
    def _fin():
        s1_ref[0] = acc_ref[0]
        s2_ref[0] = jnp.sum(colabs_ref[...])


def _run(points_i, points_j_t):
    ni = points_i.shape[0]
    nj = points_j_t.shape[1]
    k = ni // _BI
    out_shapes = (
        jax.ShapeDtypeStruct((ni, 1), jnp.int32),
        jax.ShapeDtypeStruct((1, nj), jnp.int32),
        jax.ShapeDtypeStruct((1,), jnp.float32),
        jax.ShapeDtypeStruct((1,), jnp.float32),
    )
    return pl.pallas_call(
        _nn_kernel,
        grid=(k,),
        in_specs=[
            pl.BlockSpec((_BI, 2), lambda i: (i, 0)),
            pl.BlockSpec((2, nj), lambda i: (0, 0)),
        ],
        out_specs=(
            pl.BlockSpec((_BI, 1), lambda i: (i, 0)),
            pl.BlockSpec((1, nj), lambda i: (0, 0)),
            pl.BlockSpec(memory_space=pltpu.SMEM),
            pl.BlockSpec(memory_space=pltpu.SMEM),
        ),
        out_shape=out_shapes,
        scratch_shapes=[
            pltpu.VMEM((1, nj), jnp.float32),
            pltpu.VMEM((1, nj), jnp.float32),
            pltpu.SMEM((1,), jnp.float32),
        ],
    )(points_i, points_j_t)


def kernel(pointsI, pointsJ, wi=1.0, wj=1.0):
    ni = pointsI.shape[0]
    nj = pointsJ.shape[0]
    idx1, idx2, s1, s2 = _run(pointsI, pointsJ.T)
    error = (s1[0] * wi / (2.0 * ni * ni)
             + s2[0] * wj / (2.0 * nj * nj))
    return error, idx1.reshape(ni), idx2.reshape(nj)


# speedup vs baseline: 1.1010x; 1.0649x over previous
import jax
import jax.numpy as jnp
from jax.experimental import pallas as pl
from jax.experimental.pallas import tpu as pltpu

_BI = 512


def _nn_kernel(iblk_ref, jt_ref, idx1_ref, idx2_ref, s1_ref, s2_ref,
               colmin_ref, colabs_ref, acc_ref):
    i = pl.program_id(0)
    nsteps = pl.num_programs(0)
    bi = iblk_ref.shape[0]
    n = jt_ref.shape[1]

    iblk = iblk_ref[...]
    jt = jt_ref[...]
    xi = iblk[:, 0:1]
    yi = iblk[:, 1:2]
    xj = jt[0:1, :]
    yj = jt[1:2, :]

    a2 = jnp.sum(iblk * iblk, axis=1, keepdims=True)
    b2 = jnp.sum(jt * jt, axis=0, keepdims=True)
    mm = jax.lax.dot_general(iblk, jt, (((1,), (0,)), ((), ())),
                             preferred_element_type=jnp.float32)
    dd = (a2 + b2) - 2.0 * mm
    d2 = jnp.sqrt(jnp.maximum(dd, 0.0))

    jcol = jax.lax.broadcasted_iota(jnp.int32, (bi, n), 1)
    irow = jax.lax.broadcasted_iota(jnp.int32, (bi, n), 0)

    rmin = jnp.min(d2, axis=1, keepdims=True)
    idx1 = jnp.min(jnp.where(d2 == rmin, jcol, n), axis=1, keepdims=True)
    sel1 = jcol == idx1
    mjx = jnp.sum(jnp.where(sel1, xj, 0.0), axis=1, keepdims=True)
    mjy = jnp.sum(jnp.where(sel1, yj, 0.0), axis=1, keepdims=True)
    rowabs = jnp.abs(xi - mjx) + jnp.abs(yi - mjy)
    idx1_ref[...] = idx1

    cmin = jnp.min(d2, axis=0, keepdims=True)
    cloc = jnp.min(jnp.where(d2 == cmin, irow, bi), axis=0, keepdims=True)
    sel2 = irow == cloc
    mix = jnp.sum(jnp.where(sel2, xi, 0.0), axis=0, keepdims=True)
    miy = jnp.sum(jnp.where(sel2, yi, 0.0), axis=0, keepdims=True)
    cabs = jnp.abs(xj - mix) + jnp.abs(yj - miy)
    cidx = cloc + i * bi

    @pl.when(i == 0)
    def _init():
        colmin_ref[...] = jnp.full((1, n), jnp.inf, jnp.float32)
        idx2_ref[...] = jnp.zeros((1, n), jnp.int32)
        colabs_ref[...] = jnp.zeros((1, n), jnp.float32)
        acc_ref[0] = 0.0

    upd = cmin < colmin_ref[...]
    colmin_ref[...] = jnp.where(upd, cmin, colmin_ref[...])
    idx2_ref[...] = jnp.where(upd, cidx, idx2_ref[...])
    colabs_ref[...] = jnp.where(upd, cabs, colabs_ref[...])
    acc_ref[0] += jnp.sum(rowabs)

    @pl.when(i == nsteps - 1)
    def _fin():
        s1_ref[0] = acc_ref[0]
        s2_ref[0] = jnp.sum(colabs_ref[...])


def _run(points_i, points_j_t):
    ni = points_i.shape[0]
    nj = points_j_t.shape[1]
    k = ni // _BI
    out_shapes = (
        jax.ShapeDtypeStruct((ni, 1), jnp.int32),
        jax.ShapeDtypeStruct((1, nj), jnp.int32),
        jax.ShapeDtypeStruct((1,), jnp.float32),
        jax.ShapeDtypeStruct((1,), jnp.float32),
    )
    return pl.pallas_call(
        _nn_kernel,
        grid=(k,),
        in_specs=[
            pl.BlockSpec((_BI, 2), lambda i: (i, 0)),
            pl.BlockSpec((2, nj), lambda i: (0, 0)),
        ],
        out_specs=(
            pl.BlockSpec((_BI, 1), lambda i: (i, 0)),
            pl.BlockSpec((1, nj), lambda i: (0, 0)),
            pl.BlockSpec(memory_space=pltpu.SMEM),
            pl.BlockSpec(memory_space=pltpu.SMEM),
        ),
        out_shape=out_shapes,
        scratch_shapes=[
            pltpu.VMEM((1, nj), jnp.float32),
            pltpu.VMEM((1, nj), jnp.float32),
            pltpu.SMEM((1,), jnp.float32),
        ],
    )(points_i, points_j_t)


def kernel(pointsI, pointsJ, wi=1.0, wj=1.0):
    ni = pointsI.shape[0]
    nj = pointsJ.shape[0]
    idx1, idx2, s1, s2 = _run(pointsI, pointsJ.T)
    error = (s1[0] * wi / (2.0 * ni * ni)
             + s2[0] * wj / (2.0 * nj * nj))
    return error, idx1.reshape(ni), idx2.reshape(nj)
